# Initial kernel scaffold; baseline (speedup 1.0000x reference)
import jax
import jax.numpy as jnp
from jax.experimental import pallas as pl


def kernel(x, edge_index, batch, W1_rel, b1, W1_root, W2_rel, b2, W2_root, W3_rel, b3, W3_root, Wm1, bm1, Wm2, bm2, Wm3, bm3):
    raise NotImplementedError("write your pallas kernel here")


# speedup vs baseline: 1.0334x; 1.0334x over previous
import functools

import jax
import jax.numpy as jnp
from jax.experimental import pallas as pl
from jax.experimental.pallas import tpu as pltpu

_N = 10000
_E = 320000
_G = 64


def _dense_body(agg_ref, x_ref, wr_ref, wx_ref, b_ref, o_ref):
    acc = jnp.dot(agg_ref[...], wr_ref[...], preferred_element_type=jnp.float32)
    acc = acc + jnp.dot(x_ref[...], wx_ref[...], preferred_element_type=jnp.float32)
    o_ref[...] = jnp.maximum(acc + b_ref[...], 0.0)


def _dense(agg, x, w_rel, b, w_root):
    n, f = x.shape
    o = w_rel.shape[1]
    bn = 400
    return pl.pallas_call(
        _dense_body,
        grid=(n // bn,),
        in_specs=[
            pl.BlockSpec((bn, f), lambda i: (i, 0)),
            pl.BlockSpec((bn, f), lambda i: (i, 0)),
            pl.BlockSpec((f, o), lambda i: (0, 0)),
            pl.BlockSpec((f, o), lambda i: (0, 0)),
            pl.BlockSpec((1, o), lambda i: (0, 0)),
        ],
        out_specs=pl.BlockSpec((bn, o), lambda i: (i, 0)),
        out_shape=jax.ShapeDtypeStruct((n, o), jnp.float32),
    )(agg, x, w_rel, w_root, b.reshape(1, -1))


def _head_body(h_ref, batch_ref, wm1_ref, bm1_ref, wm2_ref, bm2_ref,
               wm3_ref, bm3_ref, o_ref, pooled_ref, cnt_ref):
    i = pl.program_id(0)
    nsteps = pl.num_programs(0)

    @pl.when(i == 0)
    def _init():
        pooled_ref[...] = jnp.zeros_like(pooled_ref)
        cnt_ref[...] = jnp.zeros_like(cnt_ref)

    bids = batch_ref[0, 0, :]
    gids = jax.lax.broadcasted_iota(jnp.int32, (_G, bids.shape[0]), 0)
    onehot = (gids == bids[None, :]).astype(jnp.float32)
    pooled_ref[...] += jnp.dot(onehot, h_ref[...],
                               preferred_element_type=jnp.float32)
    cnt_ref[...] += jnp.sum(onehot, axis=1, keepdims=True)

    @pl.when(i == nsteps - 1)
    def _final():
        cnt = jnp.maximum(cnt_ref[...], 1.0)
        h = pooled_ref[...] / cnt
        h = jnp.maximum(jnp.dot(h, wm1_ref[...],
                                preferred_element_type=jnp.float32)
                        + bm1_ref[...], 0.0)
        h = jnp.maximum(jnp.dot(h, wm2_ref[...],
                                preferred_element_type=jnp.float32)
                        + bm2_ref[...], 0.0)
        logits = jnp.dot(h, wm3_ref[...],
                         preferred_element_type=jnp.float32) + bm3_ref[...]
        m = jnp.max(logits, axis=-1, keepdims=True)
        z = logits - m
        lse = jnp.log(jnp.sum(jnp.exp(z), axis=-1, keepdims=True))
        o_ref[...] = z - lse


def _head(h, batch, wm1, bm1, wm2, bm2, wm3, bm3):
    n, f = h.shape
    bn = 400
    nsteps = n // bn
    batch3 = batch.reshape(nsteps, 1, bn)
    c = wm3.shape[1]
    h1 = wm1.shape[1]
    h2 = wm2.shape[1]
    return pl.pallas_call(
        _head_body,
        grid=(nsteps,),
        in_specs=[
            pl.BlockSpec((bn, f), lambda i: (i, 0)),
            pl.BlockSpec((1, 1, bn), lambda i: (i, 0, 0)),
            pl.BlockSpec((f, h1), lambda i: (0, 0)),
            pl.BlockSpec((1, h1), lambda i: (0, 0)),
            pl.BlockSpec((h1, h2), lambda i: (0, 0)),
            pl.BlockSpec((1, h2), lambda i: (0, 0)),
            pl.BlockSpec((h2, c), lambda i: (0, 0)),
            pl.BlockSpec((1, c), lambda i: (0, 0)),
        ],
        out_specs=pl.BlockSpec((_G, c), lambda i: (0, 0)),
        out_shape=jax.ShapeDtypeStruct((_G, c), jnp.float32),
        scratch_shapes=[
            pltpu.VMEM((_G, f), jnp.float32),
            pltpu.VMEM((_G, 1), jnp.float32),
        ],
    )(h, batch3, wm1, bm1.reshape(1, -1), wm2, bm2.reshape(1, -1),
      wm3, bm3.reshape(1, -1))


def kernel(x, edge_index, batch,
           W1_rel, b1, W1_root,
           W2_rel, b2, W2_root,
           W3_rel, b3, W3_root,
           Wm1, bm1, Wm2, bm2, Wm3, bm3):
    src = edge_index[0]
    dst = edge_index[1]

    def agg(h):
        return jax.ops.segment_sum(jnp.take(h, src, axis=0), dst,
                                   num_segments=_N)

    h = _dense(agg(x), x, W1_rel, b1, W1_root)
    h = _dense(agg(h), h, W2_rel, b2, W2_root)
    h = _dense(agg(h), h, W3_rel, b3, W3_root)
    return _head(h, batch, Wm1, bm1, Wm2, bm2, Wm3, bm3)
